# baseline (speedup 1.0000x reference)
import functools

import jax
import jax.numpy as jnp
import numpy as np
from jax.experimental import pallas as pl
from jax.experimental.pallas import tpu as pltpu

_NUM_EXPERTS = 64
_CLAMP_MAX = float(np.log(100.0))


def _router_body(x_ref, w_ref, b_ref, sim_ref, temp_ref,
                 mask_ref, prob_ref, imp_ref, load_ref):
    x_t = x_ref[...]
    w = w_ref[...]
    projT = jax.lax.dot_general(
        w, x_t, (((1,), (1,)), ((), ())),
        preferred_element_type=jnp.float32) + b_ref[...]

    norm = jnp.sqrt(jnp.sum(projT * projT, axis=0, keepdims=True))
    projn = projT / jnp.maximum(norm, 1e-12)

    s = sim_ref[...]
    s_norm = jnp.sqrt(jnp.sum(s * s, axis=0, keepdims=True))
    sn = s / jnp.maximum(s_norm, 1e-12)

    scale = jnp.exp(jnp.minimum(temp_ref[0, 0], _CLAMP_MAX))
    logits = jax.lax.dot_general(
        sn, projn, (((0,), (0,)), ((), ())),
        preferred_element_type=jnp.float32) * scale

    m1 = jnp.max(logits, axis=0, keepdims=True)
    l2 = jnp.where(logits == m1, -jnp.inf, logits)
    m2 = jnp.max(l2, axis=0, keepdims=True)
    mask = (logits >= m2).astype(jnp.float32)

    p = jnp.exp(logits - m1)
    p = p / jnp.sum(p, axis=0, keepdims=True)

    mask_ref[...] = mask
    prob_ref[...] = p

    imp_part = jnp.broadcast_to(
        jnp.sum(p, axis=1, keepdims=True), (_NUM_EXPERTS, 128))
    load_part = jnp.broadcast_to(
        jnp.sum(mask, axis=1, keepdims=True), (_NUM_EXPERTS, 128))

    @pl.when(pl.program_id(0) == 0)
    def _init():
        imp_ref[...] = imp_part
        load_ref[...] = load_part

    @pl.when(pl.program_id(0) != 0)
    def _accum():
        imp_ref[...] += imp_part
        load_ref[...] += load_part


@functools.partial(jax.jit, static_argnames=())
def kernel(x, W_proj, b_proj, sim_matrix, temperature):
    n, d = x.shape
    e = sim_matrix.shape[1]
    h = sim_matrix.shape[0]
    tile = 2048
    while n % tile:
        tile //= 2
    grid = (n // tile,)

    b2 = b_proj.reshape(e, 1)
    t2 = temperature.reshape(1, 1)

    maskT, probT, imp, load = pl.pallas_call(
        _router_body,
        grid=grid,
        in_specs=[
            pl.BlockSpec((tile, d), lambda i: (i, 0)),
            pl.BlockSpec((e, d), lambda i: (0, 0)),
            pl.BlockSpec((e, 1), lambda i: (0, 0)),
            pl.BlockSpec((h, e), lambda i: (0, 0)),
            pl.BlockSpec((1, 1), lambda i: (0, 0)),
        ],
        out_specs=[
            pl.BlockSpec((e, tile), lambda i: (0, i)),
            pl.BlockSpec((e, tile), lambda i: (0, i)),
            pl.BlockSpec((e, 128), lambda i: (0, 0)),
            pl.BlockSpec((e, 128), lambda i: (0, 0)),
        ],
        out_shape=[
            jax.ShapeDtypeStruct((e, n), jnp.float32),
            jax.ShapeDtypeStruct((e, n), jnp.float32),
            jax.ShapeDtypeStruct((e, 128), jnp.float32),
            jax.ShapeDtypeStruct((e, 128), jnp.float32),
        ],
        compiler_params=pltpu.CompilerParams(
            dimension_semantics=("arbitrary",)),
    )(x, W_proj, b2, sim_matrix, t2)

    return (maskT.T, probT.T, imp[:, 0], load[:, 0])


# speedup vs baseline: 1.0137x; 1.0137x over previous
import functools

import jax
import jax.numpy as jnp
import numpy as np
from jax.experimental import pallas as pl
from jax.experimental.pallas import tpu as pltpu

_NUM_EXPERTS = 64
_CLAMP_MAX = float(np.log(100.0))


def _router_body(x_ref, w_ref, b_ref, sim_ref, temp_ref,
                 mask_ref, prob_ref, imp_ref, load_ref):
    x_t = x_ref[...]
    w = w_ref[...]
    projT = jax.lax.dot_general(
        w, x_t, (((1,), (1,)), ((), ())),
        preferred_element_type=jnp.float32) + b_ref[...]

    norm = jnp.sqrt(jnp.sum(projT * projT, axis=0, keepdims=True))
    projn = projT / jnp.maximum(norm, 1e-12)

    s = sim_ref[...]
    s_norm = jnp.sqrt(jnp.sum(s * s, axis=0, keepdims=True))
    sn = s / jnp.maximum(s_norm, 1e-12)

    scale = jnp.exp(jnp.minimum(temp_ref[0, 0], _CLAMP_MAX))
    logits = jax.lax.dot_general(
        sn, projn, (((0,), (0,)), ((), ())),
        preferred_element_type=jnp.float32) * scale

    m1 = jnp.max(logits, axis=0, keepdims=True)
    l2 = jnp.where(logits == m1, -jnp.inf, logits)
    m2 = jnp.max(l2, axis=0, keepdims=True)
    mask = (logits >= m2).astype(jnp.float32)

    p = jnp.exp(logits - m1)
    p = p / jnp.sum(p, axis=0, keepdims=True)

    mask_ref[...] = mask
    prob_ref[...] = p

    imp_part = jnp.broadcast_to(
        jnp.sum(p, axis=1, keepdims=True), (_NUM_EXPERTS, 128))
    load_part = jnp.broadcast_to(
        jnp.sum(mask, axis=1, keepdims=True), (_NUM_EXPERTS, 128))

    @pl.when(pl.program_id(0) == 0)
    def _init():
        imp_ref[...] = imp_part
        load_ref[...] = load_part

    @pl.when(pl.program_id(0) != 0)
    def _accum():
        imp_ref[...] += imp_part
        load_ref[...] += load_part


@functools.partial(jax.jit, static_argnames=())
def kernel(x, W_proj, b_proj, sim_matrix, temperature):
    n, d = x.shape
    e = sim_matrix.shape[1]
    h = sim_matrix.shape[0]
    tile = 8192
    while n % tile:
        tile //= 2
    grid = (n // tile,)

    b2 = b_proj.reshape(e, 1)
    t2 = temperature.reshape(1, 1)

    maskT, probT, imp, load = pl.pallas_call(
        _router_body,
        grid=grid,
        in_specs=[
            pl.BlockSpec((tile, d), lambda i: (i, 0)),
            pl.BlockSpec((e, d), lambda i: (0, 0)),
            pl.BlockSpec((e, 1), lambda i: (0, 0)),
            pl.BlockSpec((h, e), lambda i: (0, 0)),
            pl.BlockSpec((1, 1), lambda i: (0, 0)),
        ],
        out_specs=[
            pl.BlockSpec((e, tile), lambda i: (0, i)),
            pl.BlockSpec((e, tile), lambda i: (0, i)),
            pl.BlockSpec((e, 128), lambda i: (0, 0)),
            pl.BlockSpec((e, 128), lambda i: (0, 0)),
        ],
        out_shape=[
            jax.ShapeDtypeStruct((e, n), jnp.float32),
            jax.ShapeDtypeStruct((e, n), jnp.float32),
            jax.ShapeDtypeStruct((e, 128), jnp.float32),
            jax.ShapeDtypeStruct((e, 128), jnp.float32),
        ],
        compiler_params=pltpu.CompilerParams(
            dimension_semantics=("arbitrary",)),
    )(x, W_proj, b2, sim_matrix, t2)

    return (maskT.T, probT.T, imp[:, 0], load[:, 0])


# speedup vs baseline: 1.0563x; 1.0420x over previous
import functools

import jax
import jax.numpy as jnp
import numpy as np
from jax.experimental import pallas as pl
from jax.experimental.pallas import tpu as pltpu

_NUM_EXPERTS = 64
_CLAMP_MAX = float(np.log(100.0))


def _router_body(x_ref, w_ref, b_ref, sim_ref, temp_ref,
                 mask_ref, prob_ref, imp_ref, load_ref):
    x_t = x_ref[...]
    w = w_ref[...]
    projT = jax.lax.dot_general(
        w, x_t, (((1,), (1,)), ((), ())),
        preferred_element_type=jnp.float32) + b_ref[...]

    norm = jnp.sqrt(jnp.sum(projT * projT, axis=0, keepdims=True))
    projn = projT / jnp.maximum(norm, 1e-12)

    s = sim_ref[...]
    s_norm = jnp.sqrt(jnp.sum(s * s, axis=0, keepdims=True))
    sn = s / jnp.maximum(s_norm, 1e-12)

    scale = jnp.exp(jnp.minimum(temp_ref[0, 0], _CLAMP_MAX))
    logits = jax.lax.dot_general(
        sn, projn, (((0,), (0,)), ((), ())),
        preferred_element_type=jnp.float32) * scale

    m1 = jnp.max(logits, axis=0, keepdims=True)
    l2 = jnp.where(logits == m1, -jnp.inf, logits)
    m2 = jnp.max(l2, axis=0, keepdims=True)
    mask = (logits >= m2).astype(jnp.float32)

    p = jnp.exp(logits - m1)
    p = p / jnp.sum(p, axis=0, keepdims=True)

    mask_ref[...] = mask
    prob_ref[...] = p

    imp_part = jnp.broadcast_to(
        jnp.sum(p, axis=1, keepdims=True), (_NUM_EXPERTS, 128))
    load_part = jnp.broadcast_to(
        jnp.sum(mask, axis=1, keepdims=True), (_NUM_EXPERTS, 128))

    @pl.when(pl.program_id(0) == 0)
    def _init():
        imp_ref[...] = imp_part
        load_ref[...] = load_part

    @pl.when(pl.program_id(0) != 0)
    def _accum():
        imp_ref[...] += imp_part
        load_ref[...] += load_part


@functools.partial(jax.jit, static_argnames=())
def kernel(x, W_proj, b_proj, sim_matrix, temperature):
    n, d = x.shape
    e = sim_matrix.shape[1]
    h = sim_matrix.shape[0]
    tile = 4096
    while n % tile:
        tile //= 2
    grid = (n // tile,)

    b2 = b_proj.reshape(e, 1)
    t2 = temperature.reshape(1, 1)

    maskT, probT, imp, load = pl.pallas_call(
        _router_body,
        grid=grid,
        in_specs=[
            pl.BlockSpec((tile, d), lambda i: (i, 0)),
            pl.BlockSpec((e, d), lambda i: (0, 0)),
            pl.BlockSpec((e, 1), lambda i: (0, 0)),
            pl.BlockSpec((h, e), lambda i: (0, 0)),
            pl.BlockSpec((1, 1), lambda i: (0, 0)),
        ],
        out_specs=[
            pl.BlockSpec((e, tile), lambda i: (0, i)),
            pl.BlockSpec((e, tile), lambda i: (0, i)),
            pl.BlockSpec((e, 128), lambda i: (0, 0)),
            pl.BlockSpec((e, 128), lambda i: (0, 0)),
        ],
        out_shape=[
            jax.ShapeDtypeStruct((e, n), jnp.float32),
            jax.ShapeDtypeStruct((e, n), jnp.float32),
            jax.ShapeDtypeStruct((e, 128), jnp.float32),
            jax.ShapeDtypeStruct((e, 128), jnp.float32),
        ],
        compiler_params=pltpu.CompilerParams(
            dimension_semantics=("arbitrary",)),
    )(x, W_proj, b2, sim_matrix, t2)

    return (maskT.T, probT.T, imp[:, 0], load[:, 0])


# speedup vs baseline: 1.1335x; 1.0730x over previous
import functools

import jax
import jax.numpy as jnp
import numpy as np
from jax.experimental import pallas as pl
from jax.experimental.pallas import tpu as pltpu

_NUM_EXPERTS = 64
_CLAMP_MAX = float(np.log(100.0))


def _router_body(x_ref, w_ref, b_ref, sim_ref, temp_ref,
                 mask_ref, prob_ref, imp_ref, load_ref):
    x_t = x_ref[...]
    w = w_ref[...]
    b_col = b_ref[...].reshape(_NUM_EXPERTS, 1)
    projT = jax.lax.dot_general(
        w, x_t, (((1,), (1,)), ((), ())),
        preferred_element_type=jnp.float32) + b_col

    norm = jnp.sqrt(jnp.sum(projT * projT, axis=0, keepdims=True))
    projn = projT / jnp.maximum(norm, 1e-12)

    s = sim_ref[...]
    s_norm = jnp.sqrt(jnp.sum(s * s, axis=0, keepdims=True))
    sn = s / jnp.maximum(s_norm, 1e-12)

    scale = jnp.exp(jnp.minimum(temp_ref[0, 0], _CLAMP_MAX))
    logits = jax.lax.dot_general(
        sn, projn, (((0,), (0,)), ((), ())),
        preferred_element_type=jnp.float32) * scale

    m1 = jnp.max(logits, axis=0, keepdims=True)
    l2 = jnp.where(logits == m1, -jnp.inf, logits)
    m2 = jnp.max(l2, axis=0, keepdims=True)
    mask = (logits >= m2).astype(jnp.float32)

    p = jnp.exp(logits - m1)
    p = p / jnp.sum(p, axis=0, keepdims=True)

    mask_ref[...] = mask
    prob_ref[...] = p

    ones_row = jnp.ones((1, p.shape[1]), jnp.float32)
    imp_part = jax.lax.dot_general(
        ones_row, p, (((1,), (1,)), ((), ())),
        preferred_element_type=jnp.float32)
    load_part = jax.lax.dot_general(
        ones_row, mask, (((1,), (1,)), ((), ())),
        preferred_element_type=jnp.float32)

    @pl.when(pl.program_id(0) == 0)
    def _init():
        imp_ref[...] = imp_part
        load_ref[...] = load_part

    @pl.when(pl.program_id(0) != 0)
    def _accum():
        imp_ref[...] += imp_part
        load_ref[...] += load_part


@functools.partial(jax.jit, static_argnames=())
def kernel(x, W_proj, b_proj, sim_matrix, temperature):
    n, d = x.shape
    e = sim_matrix.shape[1]
    h = sim_matrix.shape[0]
    tile = 4096
    while n % tile:
        tile //= 2
    grid = (n // tile,)

    b2 = b_proj.reshape(1, e)
    t2 = temperature.reshape(1, 1)

    maskT, probT, imp, load = pl.pallas_call(
        _router_body,
        grid=grid,
        in_specs=[
            pl.BlockSpec((tile, d), lambda i: (i, 0)),
            pl.BlockSpec((e, d), lambda i: (0, 0)),
            pl.BlockSpec((1, e), lambda i: (0, 0)),
            pl.BlockSpec((h, e), lambda i: (0, 0)),
            pl.BlockSpec((1, 1), lambda i: (0, 0)),
        ],
        out_specs=[
            pl.BlockSpec((e, tile), lambda i: (0, i)),
            pl.BlockSpec((e, tile), lambda i: (0, i)),
            pl.BlockSpec((1, e), lambda i: (0, 0)),
            pl.BlockSpec((1, e), lambda i: (0, 0)),
        ],
        out_shape=[
            jax.ShapeDtypeStruct((e, n), jnp.float32),
            jax.ShapeDtypeStruct((e, n), jnp.float32),
            jax.ShapeDtypeStruct((1, e), jnp.float32),
            jax.ShapeDtypeStruct((1, e), jnp.float32),
        ],
        compiler_params=pltpu.CompilerParams(
            dimension_semantics=("arbitrary",)),
    )(x, W_proj, b2, sim_matrix, t2)

    return (maskT.T, probT.T, imp.reshape(e), load.reshape(e))


# speedup vs baseline: 1.1376x; 1.0036x over previous
import functools

import jax
import jax.numpy as jnp
import numpy as np
from jax.experimental import pallas as pl
from jax.experimental.pallas import tpu as pltpu

_NUM_EXPERTS = 64
_CLAMP_MAX = float(np.log(100.0))


def _router_body(x_ref, w_ref, b_ref, sim_ref, temp_ref,
                 mask_ref, prob_ref, imp_ref, load_ref):
    x_t = x_ref[...]
    w = w_ref[...]
    b_col = b_ref[...].reshape(_NUM_EXPERTS, 1)
    projT = jax.lax.dot_general(
        w, x_t, (((1,), (1,)), ((), ())),
        preferred_element_type=jnp.float32) + b_col

    norm = jnp.sqrt(jnp.sum(projT * projT, axis=0, keepdims=True))
    projn = projT / jnp.maximum(norm, 1e-12)

    s = sim_ref[...]
    s_norm = jnp.sqrt(jnp.sum(s * s, axis=0, keepdims=True))
    sn = s / jnp.maximum(s_norm, 1e-12)

    scale = jnp.exp(jnp.minimum(temp_ref[0, 0], _CLAMP_MAX))
    logits = jax.lax.dot_general(
        sn, projn, (((0,), (0,)), ((), ())),
        preferred_element_type=jnp.float32) * scale

    m1 = jnp.max(logits, axis=0, keepdims=True)
    l2 = jnp.where(logits == m1, -jnp.inf, logits)
    m2 = jnp.max(l2, axis=0, keepdims=True)
    mask = (logits >= m2).astype(jnp.float32)

    p = jnp.exp(logits - m1)
    p = p / jnp.sum(p, axis=0, keepdims=True)

    mask_ref[...] = mask
    prob_ref[...] = p

    ones_row = jnp.ones((1, p.shape[1]), jnp.float32)
    imp_part = jax.lax.dot_general(
        ones_row, p, (((1,), (1,)), ((), ())),
        preferred_element_type=jnp.float32)
    load_part = jax.lax.dot_general(
        ones_row, mask, (((1,), (1,)), ((), ())),
        preferred_element_type=jnp.float32)

    @pl.when(pl.program_id(0) == 0)
    def _init():
        imp_ref[...] = imp_part
        load_ref[...] = load_part

    @pl.when(pl.program_id(0) != 0)
    def _accum():
        imp_ref[...] += imp_part
        load_ref[...] += load_part


@functools.partial(jax.jit, static_argnames=())
def kernel(x, W_proj, b_proj, sim_matrix, temperature):
    n, d = x.shape
    e = sim_matrix.shape[1]
    h = sim_matrix.shape[0]
    tile = 4096
    while n % tile:
        tile //= 2
    grid = (n // tile,)

    b2 = b_proj.reshape(1, e)
    t2 = temperature.reshape(1, 1)

    maskT, probT, imp, load = pl.pallas_call(
        _router_body,
        grid=grid,
        in_specs=[
            pl.BlockSpec((tile, d), lambda i: (i, 0)),
            pl.BlockSpec((e, d), lambda i: (0, 0)),
            pl.BlockSpec((1, e), lambda i: (0, 0)),
            pl.BlockSpec((h, e), lambda i: (0, 0)),
            pl.BlockSpec(memory_space=pltpu.SMEM),
        ],
        out_specs=[
            pl.BlockSpec((e, tile), lambda i: (0, i)),
            pl.BlockSpec((e, tile), lambda i: (0, i)),
            pl.BlockSpec((1, e), lambda i: (0, 0)),
            pl.BlockSpec((1, e), lambda i: (0, 0)),
        ],
        out_shape=[
            jax.ShapeDtypeStruct((e, n), jnp.float32),
            jax.ShapeDtypeStruct((e, n), jnp.float32),
            jax.ShapeDtypeStruct((1, e), jnp.float32),
            jax.ShapeDtypeStruct((1, e), jnp.float32),
        ],
        compiler_params=pltpu.CompilerParams(
            dimension_semantics=("arbitrary",)),
    )(x, W_proj, b2, sim_matrix, t2)

    return (maskT.T, probT.T, imp.reshape(e), load.reshape(e))
